# baseline (speedup 1.0000x reference)
import functools

import jax
import jax.numpy as jnp
from jax import lax
from jax.experimental import pallas as pl
from jax.experimental.pallas import tpu as pltpu
from jax.experimental.pallas import tpu_sc as plsc

N = 10000
E = 320000
D = 192
DP = 256

NC = 2
NS = 16
L = 16
NW = NC * NS
EPW = E // NW
CHUNK = 80
NCHUNK = EPW // CHUNK
RBLK = 80
NBLK = N // RBLK
BPT = (NBLK + NS - 1) // NS

_SC_MESH = plsc.VectorSubcoreMesh(
    core_axis_name="c", subcore_axis_name="s", num_cores=NC, num_subcores=NS
)


@functools.partial(
    pl.kernel,
    out_type=jax.ShapeDtypeStruct((E, D), jnp.float32),
    mesh=_SC_MESH,
    scratch_types=[
        pltpu.VMEM((CHUNK,), jnp.int32),
        pltpu.VMEM((CHUNK,), jnp.int32),
        pltpu.VMEM((CHUNK,), jnp.int32),
        pltpu.VMEM((CHUNK,), jnp.int32),
        pltpu.VMEM((CHUNK, DP), jnp.float32),
        pltpu.VMEM((CHUNK, DP), jnp.float32),
        pltpu.VMEM((CHUNK, DP), jnp.float32),
        pltpu.VMEM((CHUNK, DP), jnp.float32),
        pltpu.VMEM((CHUNK, D), jnp.float32),
        pltpu.VMEM((CHUNK, D), jnp.float32),
        pltpu.SemaphoreType.DMA,
        pltpu.SemaphoreType.DMA,
        pltpu.SemaphoreType.DMA,
        pltpu.SemaphoreType.DMA,
    ],
)
def _gather_mul(src_hbm, dst_hbm, h_hbm, h2_hbm,
                ixs0, ixd0, ixs1, ixd1, rs0, rd0, rs1, rd1, pr0, pr1,
                gs0, gs1, ws0, ws1):
    cid = lax.axis_index("c")
    sid = lax.axis_index("s")
    wid = sid * NC + cid
    base0 = wid * EPW

    sets = ((ixs0, ixd0, rs0, rd0, pr0, gs0, ws0),
            (ixs1, ixd1, rs1, rd1, pr1, gs1, ws1))

    def issue(c, b):
        ixs, ixd, rs, rd, _, gs, _ = sets[b]
        base = base0 + c * CHUNK
        pltpu.sync_copy(src_hbm.at[pl.ds(base, CHUNK)], ixs)
        pltpu.sync_copy(dst_hbm.at[pl.ds(base, CHUNK)], ixd)
        pltpu.async_copy(h_hbm.at[ixs], rs, gs)
        pltpu.async_copy(h_hbm.at[ixd], rd, gs)

    def step(c, b, last):
        ixs, ixd, rs, rd, pr, gs, ws = sets[b]
        if not last:
            issue(c + 1, 1 - b)
        pltpu.make_async_copy(h_hbm.at[ixs], rs, gs).wait()
        pltpu.make_async_copy(h_hbm.at[ixd], rd, gs).wait()

        @pl.when(c >= 2)
        def _():
            pltpu.make_async_copy(
                pr, h2_hbm.at[pl.ds(base0, CHUNK)], ws).wait()

        def edge_body(i, carry2):
            for j in range(D // L):
                sl = pl.ds(j * L, L)
                pr[i, sl] = rs[i, sl] * rd[i, sl]
            return carry2

        lax.fori_loop(0, CHUNK, edge_body, 0, unroll=False)
        pltpu.async_copy(pr, h2_hbm.at[pl.ds(base0 + c * CHUNK, CHUNK)], ws)

    issue(0, 0)

    def pair_body(k, carry):
        step(2 * k, 0, False)
        step(2 * k + 1, 1, False)
        return carry

    lax.fori_loop(0, NCHUNK // 2, pair_body, 0, unroll=False)
    step(NCHUNK - 1, 0, True)

    pltpu.make_async_copy(pr1, h2_hbm.at[pl.ds(base0, CHUNK)], ws1).wait()
    pltpu.make_async_copy(pr0, h2_hbm.at[pl.ds(base0, CHUNK)], ws0).wait()


_BE = 512


def _gate_body(h2_ref, w1t_ref, b1_ref, a_ref, w2_ref, b2_ref, e_ref,
               erep_ref):
    t = jnp.dot(h2_ref[...], w1t_ref[...], preferred_element_type=jnp.float32)
    t = t + b1_ref[...]
    a = a_ref[0, 0]
    t = jnp.where(t >= 0, t, a * t)
    g = lax.dot_general(w2_ref[...], t, (((1,), (1,)), ((), ())),
                        preferred_element_type=jnp.float32)
    e = jnp.tanh(g + b2_ref[0, 0])
    e_ref[...] = e
    erep_ref[...] = jnp.broadcast_to(e.reshape(_BE, 1), (_BE, L))


_gate = pl.pallas_call(
    _gate_body,
    grid=(E // _BE,),
    in_specs=[
        pl.BlockSpec((_BE, D), lambda i: (i, 0)),
        pl.BlockSpec((D, D), lambda i: (0, 0)),
        pl.BlockSpec((1, D), lambda i: (0, 0)),
        pl.BlockSpec((1, 1), lambda i: (0, 0)),
        pl.BlockSpec((1, D), lambda i: (0, 0)),
        pl.BlockSpec((1, 1), lambda i: (0, 0)),
    ],
    out_specs=[
        pl.BlockSpec((1, _BE), lambda i: (0, i)),
        pl.BlockSpec((_BE, L), lambda i: (i, 0)),
    ],
    out_shape=[
        jax.ShapeDtypeStruct((1, E), jnp.float32),
        jax.ShapeDtypeStruct((E, L), jnp.float32),
    ],
)


HP = 128
CH_C = 40
NCH_C = EPW // CH_C


@functools.partial(
    pl.kernel,
    out_type=jax.ShapeDtypeStruct((NC, N, HP), jnp.float32),
    mesh=_SC_MESH,
    scratch_types=[
        pltpu.VMEM((CH_C,), jnp.int32),
        pltpu.VMEM((CH_C,), jnp.int32),
        pltpu.VMEM((CH_C,), jnp.int32),
        pltpu.VMEM((CH_C,), jnp.int32),
        pltpu.VMEM((CH_C, L), jnp.float32),
        pltpu.VMEM((CH_C, L), jnp.float32),
        pltpu.VMEM((CH_C, HP), jnp.float32),
        pltpu.VMEM((CH_C, HP), jnp.float32),
        pltpu.VMEM((CH_C, HP), jnp.float32),
        pltpu.VMEM((CH_C, HP), jnp.float32),
        pltpu.VMEM_SHARED((N, HP), jnp.float32),
        pltpu.SemaphoreType.DMA,
        pltpu.SemaphoreType.DMA,
        pltpu.SemaphoreType.DMA,
        pltpu.SemaphoreType.DMA,
    ],
)
def _scatter_sum(src_hbm, dst_hbm, erep_hbm, tbl_hbm, zeros_hbm, zp_hbm,
                 ixs0, ixs1, ixd0, ixd1, em0, em1,
                 rg0, rg1, rw0, rw1, z_sh,
                 gs0, gs1, ss0, ss1):
    cid = lax.axis_index("c")
    sid = lax.axis_index("s")
    wid = sid * NC + cid
    base0 = wid * EPW

    sets = ((ixs0, ixd0, em0, rg0, rw0, gs0, ss0),
            (ixs1, ixd1, em1, rg1, rw1, gs1, ss1))

    for j in range(BPT):
        b = sid + j * NS

        @pl.when(b < NBLK)
        def _():
            pltpu.sync_copy(zeros_hbm, z_sh.at[pl.ds(b * RBLK, RBLK)])

    plsc.subcore_barrier()

    def drain_scatter(b):
        _, ixd, _, _, rw, _, ss = sets[b]
        pltpu.make_async_copy(rw, z_sh.at[ixd], ss).wait()

    def step(c, b, last):
        ixs, ixd, em, rg, rw, gs, ss = sets[b]
        nb = 1 - b
        ixs_n, ixd_n, em_n, rg_n, _, gs_n, _ = sets[nb]
        if not last:
            nbase = base0 + (c + 1) * CH_C
            pltpu.sync_copy(src_hbm.at[pl.ds(nbase, CH_C)], ixs_n)
            pltpu.sync_copy(erep_hbm.at[pl.ds(nbase, CH_C)], em_n)
            pltpu.async_copy(tbl_hbm.at[ixs_n], rg_n, gs_n)
        pltpu.make_async_copy(tbl_hbm.at[ixs], rg, gs).wait()

        def edge_body(i, carry2):
            eb = em[i, :]
            for j in range(HP // L):
                sl = pl.ds(j * L, L)
                rw[i, sl] = rg[i, sl] * eb
            return carry2

        lax.fori_loop(0, CH_C, edge_body, 0, unroll=False)

        @pl.when(c >= 1)
        def _():
            drain_scatter(nb)

        if not last:
            pltpu.sync_copy(dst_hbm.at[pl.ds(base0 + (c + 1) * CH_C, CH_C)],
                            ixd_n)
        pltpu.async_copy(rw, z_sh.at[ixd], ss, add=True)

    pltpu.sync_copy(src_hbm.at[pl.ds(base0, CH_C)], ixs0)
    pltpu.sync_copy(erep_hbm.at[pl.ds(base0, CH_C)], em0)
    pltpu.async_copy(tbl_hbm.at[ixs0], rg0, gs0)
    pltpu.sync_copy(dst_hbm.at[pl.ds(base0, CH_C)], ixd0)

    def pair_body(k, carry):
        step(2 * k, 0, False)
        step(2 * k + 1, 1, False)
        return carry

    lax.fori_loop(0, NCH_C // 2 - 1, pair_body, 0, unroll=False)
    step(NCH_C - 2, 0, False)
    step(NCH_C - 1, 1, True)
    drain_scatter(1)
    plsc.subcore_barrier()

    for j in range(BPT):
        b = sid + j * NS

        @pl.when(b < NBLK)
        def _():
            pltpu.sync_copy(z_sh.at[pl.ds(b * RBLK, RBLK)],
                            zp_hbm.at[cid, pl.ds(b * RBLK, RBLK)])


_BN = 1000


def _combine_body(lo_ref, hi_ref, z_ref):
    lo = lo_ref[0] + lo_ref[1]
    hi = hi_ref[0] + hi_ref[1]
    z_ref[...] = jnp.concatenate([lo, hi[:, : D - HP]], axis=1)


_combine = pl.pallas_call(
    _combine_body,
    grid=(N // _BN,),
    in_specs=[
        pl.BlockSpec((NC, _BN, HP), lambda i: (0, i, 0)),
        pl.BlockSpec((NC, _BN, HP), lambda i: (0, i, 0)),
    ],
    out_specs=pl.BlockSpec((_BN, D), lambda i: (i, 0)),
    out_shape=jax.ShapeDtypeStruct((N, D), jnp.float32),
)


def kernel(h, edge_index, W1, b1, a, W2, b2):
    src = edge_index[0]
    dst = edge_index[1]
    h_pad = jnp.pad(h, ((0, 0), (0, DP - D)))
    h2 = _gather_mul(src, dst, h_pad)
    e_row, e_rep = _gate(
        h2,
        W1.T,
        b1.reshape(1, D),
        jnp.asarray(a, jnp.float32).reshape(1, 1),
        W2,
        b2.reshape(1, 1),
    )
    e = e_row.reshape(E)
    zeros = jnp.zeros((RBLK, HP), jnp.float32)
    tbl2 = h_pad.reshape(2 * N, HP)
    src2 = src * 2
    zp_lo = _scatter_sum(src2, dst, e_rep, tbl2, zeros)
    zp_hi = _scatter_sum(src2 + 1, dst, e_rep, tbl2, zeros)
    z = _combine(zp_lo, zp_hi)
    return (z, e.reshape(E, 1))


# speedup vs baseline: 1.0029x; 1.0029x over previous
import functools

import jax
import jax.numpy as jnp
from jax import lax
from jax.experimental import pallas as pl
from jax.experimental.pallas import tpu as pltpu
from jax.experimental.pallas import tpu_sc as plsc

N = 10000
E = 320000
D = 192
DP = 256

NC = 2
NS = 16
L = 16
NW = NC * NS
EPW = E // NW
CHUNK = 80
NCHUNK = EPW // CHUNK
RBLK = 80
NBLK = N // RBLK
BPT = (NBLK + NS - 1) // NS

_SC_MESH = plsc.VectorSubcoreMesh(
    core_axis_name="c", subcore_axis_name="s", num_cores=NC, num_subcores=NS
)


@functools.partial(
    pl.kernel,
    out_type=jax.ShapeDtypeStruct((E, D), jnp.float32),
    mesh=_SC_MESH,
    scratch_types=[
        pltpu.VMEM((CHUNK,), jnp.int32),
        pltpu.VMEM((CHUNK,), jnp.int32),
        pltpu.VMEM((CHUNK,), jnp.int32),
        pltpu.VMEM((CHUNK,), jnp.int32),
        pltpu.VMEM((CHUNK, DP), jnp.float32),
        pltpu.VMEM((CHUNK, DP), jnp.float32),
        pltpu.VMEM((CHUNK, DP), jnp.float32),
        pltpu.VMEM((CHUNK, DP), jnp.float32),
        pltpu.VMEM((CHUNK, D), jnp.float32),
        pltpu.VMEM((CHUNK, D), jnp.float32),
        pltpu.SemaphoreType.DMA,
        pltpu.SemaphoreType.DMA,
        pltpu.SemaphoreType.DMA,
        pltpu.SemaphoreType.DMA,
    ],
)
def _gather_mul(src_hbm, dst_hbm, h_hbm, h2_hbm,
                ixs0, ixd0, ixs1, ixd1, rs0, rd0, rs1, rd1, pr0, pr1,
                gs0, gs1, ws0, ws1):
    cid = lax.axis_index("c")
    sid = lax.axis_index("s")
    wid = sid * NC + cid
    base0 = wid * EPW

    sets = ((ixs0, ixd0, rs0, rd0, pr0, gs0, ws0),
            (ixs1, ixd1, rs1, rd1, pr1, gs1, ws1))

    def issue(c, b):
        ixs, ixd, rs, rd, _, gs, _ = sets[b]
        base = base0 + c * CHUNK
        pltpu.sync_copy(src_hbm.at[pl.ds(base, CHUNK)], ixs)
        pltpu.sync_copy(dst_hbm.at[pl.ds(base, CHUNK)], ixd)
        pltpu.async_copy(h_hbm.at[ixs], rs, gs)
        pltpu.async_copy(h_hbm.at[ixd], rd, gs)

    def step(c, b, last):
        ixs, ixd, rs, rd, pr, gs, ws = sets[b]
        if not last:
            issue(c + 1, 1 - b)
        pltpu.make_async_copy(h_hbm.at[ixs], rs, gs).wait()
        pltpu.make_async_copy(h_hbm.at[ixd], rd, gs).wait()

        @pl.when(c >= 2)
        def _():
            pltpu.make_async_copy(
                pr, h2_hbm.at[pl.ds(base0, CHUNK)], ws).wait()

        def edge_body(i, carry2):
            for j in range(D // L):
                sl = pl.ds(j * L, L)
                pr[i, sl] = rs[i, sl] * rd[i, sl]
            return carry2

        lax.fori_loop(0, CHUNK, edge_body, 0, unroll=False)
        pltpu.async_copy(pr, h2_hbm.at[pl.ds(base0 + c * CHUNK, CHUNK)], ws)

    issue(0, 0)

    def pair_body(k, carry):
        step(2 * k, 0, False)
        step(2 * k + 1, 1, False)
        return carry

    lax.fori_loop(0, NCHUNK // 2, pair_body, 0, unroll=False)
    step(NCHUNK - 1, 0, True)

    pltpu.make_async_copy(pr1, h2_hbm.at[pl.ds(base0, CHUNK)], ws1).wait()
    pltpu.make_async_copy(pr0, h2_hbm.at[pl.ds(base0, CHUNK)], ws0).wait()


_BE = 512


def _gate_body(h2_ref, w1t_ref, b1_ref, a_ref, w2_ref, b2_ref, e_ref,
               erep_ref):
    t = jnp.dot(h2_ref[...], w1t_ref[...], preferred_element_type=jnp.float32)
    t = t + b1_ref[...]
    a = a_ref[0, 0]
    t = jnp.where(t >= 0, t, a * t)
    g = lax.dot_general(w2_ref[...], t, (((1,), (1,)), ((), ())),
                        preferred_element_type=jnp.float32)
    e = jnp.tanh(g + b2_ref[0, 0])
    e_ref[...] = e
    erep_ref[...] = jnp.broadcast_to(e.reshape(_BE, 1), (_BE, L))


_gate = pl.pallas_call(
    _gate_body,
    grid=(E // _BE,),
    in_specs=[
        pl.BlockSpec((_BE, D), lambda i: (i, 0)),
        pl.BlockSpec((D, D), lambda i: (0, 0)),
        pl.BlockSpec((1, D), lambda i: (0, 0)),
        pl.BlockSpec((1, 1), lambda i: (0, 0)),
        pl.BlockSpec((1, D), lambda i: (0, 0)),
        pl.BlockSpec((1, 1), lambda i: (0, 0)),
    ],
    out_specs=[
        pl.BlockSpec((1, _BE), lambda i: (0, i)),
        pl.BlockSpec((_BE, L), lambda i: (i, 0)),
    ],
    out_shape=[
        jax.ShapeDtypeStruct((1, E), jnp.float32),
        jax.ShapeDtypeStruct((E, L), jnp.float32),
    ],
)


HP = 128
CH_C = 40
EPT = E // NS
NCH_C = EPT // CH_C


@functools.partial(
    pl.kernel,
    out_type=jax.ShapeDtypeStruct((NC, N, HP), jnp.float32),
    mesh=_SC_MESH,
    scratch_types=[
        pltpu.VMEM((CH_C,), jnp.int32),
        pltpu.VMEM((CH_C,), jnp.int32),
        pltpu.VMEM((CH_C,), jnp.int32),
        pltpu.VMEM((CH_C,), jnp.int32),
        pltpu.VMEM((CH_C, L), jnp.float32),
        pltpu.VMEM((CH_C, L), jnp.float32),
        pltpu.VMEM((CH_C, HP), jnp.float32),
        pltpu.VMEM((CH_C, HP), jnp.float32),
        pltpu.VMEM((CH_C, HP), jnp.float32),
        pltpu.VMEM((CH_C, HP), jnp.float32),
        pltpu.VMEM_SHARED((N, HP), jnp.float32),
        pltpu.SemaphoreType.DMA,
        pltpu.SemaphoreType.DMA,
        pltpu.SemaphoreType.DMA,
        pltpu.SemaphoreType.DMA,
    ],
)
def _scatter_sum(srclo_hbm, srchi_hbm, dst_hbm, erep_hbm, tbl_hbm, zeros_hbm,
                 zp_hbm,
                 ixs0, ixs1, ixd0, ixd1, em0, em1,
                 rg0, rg1, rw0, rw1, z_sh,
                 gs0, gs1, ss0, ss1):
    cid = lax.axis_index("c")
    sid = lax.axis_index("s")
    base0 = sid * EPT

    def load_src(base, ixs):
        @pl.when(cid == 0)
        def _():
            pltpu.sync_copy(srclo_hbm.at[pl.ds(base, CH_C)], ixs)

        @pl.when(cid != 0)
        def _():
            pltpu.sync_copy(srchi_hbm.at[pl.ds(base, CH_C)], ixs)

    sets = ((ixs0, ixd0, em0, rg0, rw0, gs0, ss0),
            (ixs1, ixd1, em1, rg1, rw1, gs1, ss1))

    for j in range(BPT):
        b = sid + j * NS

        @pl.when(b < NBLK)
        def _():
            pltpu.sync_copy(zeros_hbm, z_sh.at[pl.ds(b * RBLK, RBLK)])

    plsc.subcore_barrier()

    def drain_scatter(b):
        _, ixd, _, _, rw, _, ss = sets[b]
        pltpu.make_async_copy(rw, z_sh.at[ixd], ss).wait()

    def step(c, b, last):
        ixs, ixd, em, rg, rw, gs, ss = sets[b]
        nb = 1 - b
        ixs_n, ixd_n, em_n, rg_n, _, gs_n, _ = sets[nb]
        if not last:
            nbase = base0 + (c + 1) * CH_C
            load_src(nbase, ixs_n)
            pltpu.sync_copy(erep_hbm.at[pl.ds(nbase, CH_C)], em_n)
            pltpu.async_copy(tbl_hbm.at[ixs_n], rg_n, gs_n)
        pltpu.make_async_copy(tbl_hbm.at[ixs], rg, gs).wait()

        def edge_body(i, carry2):
            eb = em[i, :]
            for j in range(HP // L):
                sl = pl.ds(j * L, L)
                rw[i, sl] = rg[i, sl] * eb
            return carry2

        lax.fori_loop(0, CH_C, edge_body, 0, unroll=False)

        @pl.when(c >= 1)
        def _():
            drain_scatter(nb)

        if not last:
            pltpu.sync_copy(dst_hbm.at[pl.ds(base0 + (c + 1) * CH_C, CH_C)],
                            ixd_n)
        pltpu.async_copy(rw, z_sh.at[ixd], ss, add=True)

    load_src(base0, ixs0)
    pltpu.sync_copy(erep_hbm.at[pl.ds(base0, CH_C)], em0)
    pltpu.async_copy(tbl_hbm.at[ixs0], rg0, gs0)
    pltpu.sync_copy(dst_hbm.at[pl.ds(base0, CH_C)], ixd0)

    def pair_body(k, carry):
        step(2 * k, 0, False)
        step(2 * k + 1, 1, False)
        return carry

    lax.fori_loop(0, NCH_C // 2 - 1, pair_body, 0, unroll=False)
    step(NCH_C - 2, 0, False)
    step(NCH_C - 1, 1, True)
    drain_scatter(1)
    plsc.subcore_barrier()

    for j in range(BPT):
        b = sid + j * NS

        @pl.when(b < NBLK)
        def _():
            pltpu.sync_copy(z_sh.at[pl.ds(b * RBLK, RBLK)],
                            zp_hbm.at[cid, pl.ds(b * RBLK, RBLK)])


def kernel(h, edge_index, W1, b1, a, W2, b2):
    src = edge_index[0]
    dst = edge_index[1]
    h_pad = jnp.pad(h, ((0, 0), (0, DP - D)))
    h2 = _gather_mul(src, dst, h_pad)
    e_row, e_rep = _gate(
        h2,
        W1.T,
        b1.reshape(1, D),
        jnp.asarray(a, jnp.float32).reshape(1, 1),
        W2,
        b2.reshape(1, 1),
    )
    e = e_row.reshape(E)
    zeros = jnp.zeros((RBLK, HP), jnp.float32)
    tbl2 = h_pad.reshape(2 * N, HP)
    src2 = src * 2
    zp = _scatter_sum(src2, src2 + 1, dst, e_rep, tbl2, zeros)
    z = jnp.concatenate([zp[0], zp[1, :, : D - HP]], axis=1)
    return (z, e.reshape(E, 1))
